# baseline (speedup 1.0000x reference)
import functools

import jax
import jax.numpy as jnp
from jax import lax
from jax.experimental import pallas as pl
from jax.experimental.pallas import tpu as pltpu
from jax.experimental.pallas import tpu_sc as plsc

F32 = jnp.float32
NC = 2
NS = 16
NW = NC * NS
C = 128


def _sc_mesh():
    return plsc.VectorSubcoreMesh(
        core_axis_name="c", subcore_axis_name="s",
        num_cores=NC, num_subcores=NS)


def _make_deg(n, nch):
    rpt = n // NS

    @functools.partial(
        pl.kernel,
        out_type=jax.ShapeDtypeStruct((NC, n), F32),
        mesh=_sc_mesh(),
        scratch_types=[
            pltpu.VMEM((nch, C), jnp.int32),
            pltpu.VMEM((C,), F32),
            pltpu.VMEM_SHARED((n,), F32),
        ],
    )
    def deg_kernel(dst_hbm, ones_hbm, zero_hbm, out_hbm, dstv, onesv, acc):
        cid = lax.axis_index("c")
        sid = lax.axis_index("s")
        wid = sid * NC + cid
        pltpu.sync_copy(dst_hbm.at[wid], dstv)
        pltpu.sync_copy(ones_hbm, onesv)
        pltpu.sync_copy(zero_hbm, acc.at[pl.ds(sid * rpt, rpt)])
        plsc.subcore_barrier()

        def step(j, carry):
            pltpu.sync_copy(onesv, acc.at[dstv.at[j]], add=True)
            return carry
        lax.fori_loop(0, nch, step, 0)

        plsc.subcore_barrier()
        pltpu.sync_copy(acc.at[pl.ds(sid * rpt, rpt)],
                        out_hbm.at[cid].at[pl.ds(sid * rpt, rpt)])

    return deg_kernel


def _make_msg(n, d, nch):
    rpt = n // NS

    @functools.partial(
        pl.kernel,
        out_type=jax.ShapeDtypeStruct((NC, n, d), F32),
        mesh=_sc_mesh(),
        scratch_types=[
            pltpu.VMEM((nch, C), jnp.int32),
            pltpu.VMEM((nch, C), jnp.int32),
            pltpu.VMEM((C, d), F32),
            pltpu.VMEM_SHARED((n, d), F32),
            pltpu.SemaphoreType.DMA,
        ],
    )
    def msg_kernel(hs_hbm, src_hbm, dst_hbm, zero_hbm, out_hbm,
                   srcv, dstv, rowsv, acc, sem):
        cid = lax.axis_index("c")
        sid = lax.axis_index("s")
        wid = sid * NC + cid
        pltpu.sync_copy(src_hbm.at[wid], srcv)
        pltpu.sync_copy(dst_hbm.at[wid], dstv)
        pltpu.sync_copy(zero_hbm, acc.at[pl.ds(sid * rpt, rpt)])
        plsc.subcore_barrier()

        def step(j, carry):
            pltpu.async_copy(hs_hbm.at[srcv.at[j]], rowsv, sem).wait()
            pltpu.sync_copy(rowsv, acc.at[dstv.at[j]], add=True)
            return carry
        lax.fori_loop(0, nch, step, 0)

        plsc.subcore_barrier()
        pltpu.sync_copy(acc.at[pl.ds(sid * rpt, rpt)],
                        out_hbm.at[cid].at[pl.ds(sid * rpt, rpt)])

    return msg_kernel


def _dinv_of(deg_blk):
    return lax.rsqrt(deg_blk[0] + deg_blk[1] + 1.0)


def _tc_first(degp, x, w):
    n, d = x.shape
    blk = n // 10

    def body(deg_ref, x_ref, w_ref, o_ref):
        dinv = _dinv_of(deg_ref[...])
        o_ref[...] = jnp.dot(x_ref[...], w_ref[...],
                             preferred_element_type=F32) * dinv

    return pl.pallas_call(
        body,
        grid=(n // blk,),
        in_specs=[
            pl.BlockSpec((NC, blk, 1), lambda i: (0, i, 0)),
            pl.BlockSpec((blk, d), lambda i: (i, 0)),
            pl.BlockSpec((d, d), lambda i: (0, 0)),
        ],
        out_specs=pl.BlockSpec((blk, d), lambda i: (i, 0)),
        out_shape=jax.ShapeDtypeStruct((n, d), F32),
    )(degp, x, w)


def _tc_mid(degp, p, hs, b, w):
    n, d = hs.shape
    blk = n // 10

    def body(deg_ref, p_ref, hs_ref, b_ref, w_ref, o_ref):
        dinv = _dinv_of(deg_ref[...])
        pp = p_ref[...]
        o = jnp.maximum((pp[0] + pp[1] + hs_ref[...]) * dinv + b_ref[...], 0.0)
        o_ref[...] = jnp.dot(o, w_ref[...], preferred_element_type=F32) * dinv

    return pl.pallas_call(
        body,
        grid=(n // blk,),
        in_specs=[
            pl.BlockSpec((NC, blk, 1), lambda i: (0, i, 0)),
            pl.BlockSpec((NC, blk, d), lambda i: (0, i, 0)),
            pl.BlockSpec((blk, d), lambda i: (i, 0)),
            pl.BlockSpec((1, d), lambda i: (0, 0)),
            pl.BlockSpec((d, d), lambda i: (0, 0)),
        ],
        out_specs=pl.BlockSpec((blk, d), lambda i: (i, 0)),
        out_shape=jax.ShapeDtypeStruct((n, d), F32),
    )(degp, p, hs, b, w)


def _tc_final(degp, p, hs, b):
    n, d = hs.shape
    blk = n // 10

    def body(deg_ref, p_ref, hs_ref, b_ref, o_ref):
        dinv = _dinv_of(deg_ref[...])
        pp = p_ref[...]
        o_ref[...] = jnp.maximum(
            (pp[0] + pp[1] + hs_ref[...]) * dinv + b_ref[...], 0.0)

    return pl.pallas_call(
        body,
        grid=(n // blk,),
        in_specs=[
            pl.BlockSpec((NC, blk, 1), lambda i: (0, i, 0)),
            pl.BlockSpec((NC, blk, d), lambda i: (0, i, 0)),
            pl.BlockSpec((blk, d), lambda i: (i, 0)),
            pl.BlockSpec((1, d), lambda i: (0, 0)),
        ],
        out_specs=pl.BlockSpec((blk, d), lambda i: (i, 0)),
        out_shape=jax.ShapeDtypeStruct((n, d), F32),
    )(degp, p, hs, b)


def kernel(x, edge_index, W1, b1, W2, b2, W3, b3):
    n, d = x.shape
    e = edge_index.shape[1]
    nch = (e + NW * C - 1) // (NW * C)
    ep = NW * nch * C
    n_deg = ((n + NS * 128 - 1) // (NS * 128)) * (NS * 128)
    n_acc = n_deg

    trash = n + jnp.arange(ep - e, dtype=jnp.int32) % (n_acc - n)
    src = jnp.pad(edge_index[0].astype(jnp.int32), (0, ep - e)
                  ).reshape(NW, nch, C)
    dst = jnp.concatenate([edge_index[1].astype(jnp.int32), trash]
                          ).reshape(NW, nch, C)
    ones_r = jnp.ones((C,), F32)
    zdeg = jnp.zeros((n_deg // NS,), F32)
    zmsg = jnp.zeros((n_acc // NS, d), F32)

    deg_fn = _make_deg(n_deg, nch)
    msg_fn = _make_msg(n_acc, d, nch)

    degp = deg_fn(dst, ones_r, zdeg).reshape(NC, n_deg, 1)
    hs1 = _tc_first(degp, x, W1)
    p1 = msg_fn(hs1, src, dst, zmsg)
    hs2 = _tc_mid(degp, p1, hs1, b1.reshape(1, -1), W2)
    p2 = msg_fn(hs2, src, dst, zmsg)
    hs3 = _tc_mid(degp, p2, hs2, b2.reshape(1, -1), W3)
    p3 = msg_fn(hs3, src, dst, zmsg)
    return _tc_final(degp, p3, hs3, b3.reshape(1, -1))


# speedup vs baseline: 1.6415x; 1.6415x over previous
import functools

import jax
import jax.numpy as jnp
from jax import lax
from jax.experimental import pallas as pl
from jax.experimental.pallas import tpu as pltpu
from jax.experimental.pallas import tpu_sc as plsc

F32 = jnp.float32
NC = 2
NS = 16
NW = NC * NS
C = 125


def _sc_mesh():
    return plsc.VectorSubcoreMesh(
        core_axis_name="c", subcore_axis_name="s",
        num_cores=NC, num_subcores=NS)


def _make_deg(n, nch):
    rpt = n // NS

    @functools.partial(
        pl.kernel,
        out_type=jax.ShapeDtypeStruct((NC, n), F32),
        mesh=_sc_mesh(),
        scratch_types=[
            pltpu.VMEM((nch, C), jnp.int32),
            pltpu.VMEM((C,), F32),
            pltpu.VMEM_SHARED((n,), F32),
        ],
    )
    def deg_kernel(dst_hbm, ones_hbm, zero_hbm, out_hbm, dstv, onesv, acc):
        cid = lax.axis_index("c")
        sid = lax.axis_index("s")
        wid = sid * NC + cid
        pltpu.sync_copy(dst_hbm.at[wid], dstv)
        pltpu.sync_copy(ones_hbm, onesv)
        pltpu.sync_copy(zero_hbm, acc.at[pl.ds(sid * rpt, rpt)])
        plsc.subcore_barrier()

        def step(j, carry):
            pltpu.sync_copy(onesv, acc.at[dstv.at[j]], add=True)
            return carry
        lax.fori_loop(0, nch, step, 0)

        plsc.subcore_barrier()
        pltpu.sync_copy(acc.at[pl.ds(sid * rpt, rpt)],
                        out_hbm.at[cid].at[pl.ds(sid * rpt, rpt)])

    return deg_kernel


def _make_msg(n, d, nch):
    rpt = n // NS

    @functools.partial(
        pl.kernel,
        out_type=jax.ShapeDtypeStruct((NC, n, d), F32),
        mesh=_sc_mesh(),
        scratch_types=[
            pltpu.VMEM((nch, C), jnp.int32),
            pltpu.VMEM((nch, C), jnp.int32),
            pltpu.VMEM((C, d), F32),
            pltpu.VMEM_SHARED((n, d), F32),
            pltpu.SemaphoreType.DMA,
        ],
    )
    def msg_kernel(hs_hbm, src_hbm, dst_hbm, zero_hbm, out_hbm,
                   srcv, dstv, rowsv, acc, sem):
        cid = lax.axis_index("c")
        sid = lax.axis_index("s")
        wid = sid * NC + cid
        pltpu.sync_copy(src_hbm.at[wid], srcv)
        pltpu.sync_copy(dst_hbm.at[wid], dstv)
        pltpu.sync_copy(zero_hbm, acc.at[pl.ds(sid * rpt, rpt)])
        plsc.subcore_barrier()

        def step(j, carry):
            pltpu.async_copy(hs_hbm.at[srcv.at[j]], rowsv, sem).wait()
            pltpu.sync_copy(rowsv, acc.at[dstv.at[j]], add=True)
            return carry
        lax.fori_loop(0, nch, step, 0)

        plsc.subcore_barrier()
        pltpu.sync_copy(acc.at[pl.ds(sid * rpt, rpt)],
                        out_hbm.at[cid].at[pl.ds(sid * rpt, rpt)])

    return msg_kernel


def _dinv_of(deg_blk):
    return lax.rsqrt(deg_blk[0] + deg_blk[1] + 1.0)


def _tc_first(degp, x, w):
    n, d = x.shape
    blk = n // 16

    def body(deg_ref, x_ref, w_ref, o_ref):
        dinv = _dinv_of(deg_ref[...])
        o_ref[...] = jnp.dot(x_ref[...], w_ref[...],
                             preferred_element_type=F32) * dinv

    return pl.pallas_call(
        body,
        grid=(n // blk,),
        in_specs=[
            pl.BlockSpec((NC, blk, 1), lambda i: (0, i, 0)),
            pl.BlockSpec((blk, d), lambda i: (i, 0)),
            pl.BlockSpec((d, d), lambda i: (0, 0)),
        ],
        out_specs=pl.BlockSpec((blk, d), lambda i: (i, 0)),
        out_shape=jax.ShapeDtypeStruct((n, d), F32),
    )(degp, x, w)


def _tc_mid(degp, p, hs, b, w):
    n, d = hs.shape
    blk = n // 16

    def body(deg_ref, p_ref, hs_ref, b_ref, w_ref, o_ref):
        dinv = _dinv_of(deg_ref[...])
        pp = p_ref[...]
        o = jnp.maximum((pp[0] + pp[1] + hs_ref[...]) * dinv + b_ref[...], 0.0)
        o_ref[...] = jnp.dot(o, w_ref[...], preferred_element_type=F32) * dinv

    return pl.pallas_call(
        body,
        grid=(n // blk,),
        in_specs=[
            pl.BlockSpec((NC, blk, 1), lambda i: (0, i, 0)),
            pl.BlockSpec((NC, blk, d), lambda i: (0, i, 0)),
            pl.BlockSpec((blk, d), lambda i: (i, 0)),
            pl.BlockSpec((1, d), lambda i: (0, 0)),
            pl.BlockSpec((d, d), lambda i: (0, 0)),
        ],
        out_specs=pl.BlockSpec((blk, d), lambda i: (i, 0)),
        out_shape=jax.ShapeDtypeStruct((n, d), F32),
    )(degp, p, hs, b, w)


def _tc_final(degp, p, hs, b):
    n, d = hs.shape
    blk = n // 16

    def body(deg_ref, p_ref, hs_ref, b_ref, o_ref):
        dinv = _dinv_of(deg_ref[...])
        pp = p_ref[...]
        o_ref[...] = jnp.maximum(
            (pp[0] + pp[1] + hs_ref[...]) * dinv + b_ref[...], 0.0)

    return pl.pallas_call(
        body,
        grid=(n // blk,),
        in_specs=[
            pl.BlockSpec((NC, blk, 1), lambda i: (0, i, 0)),
            pl.BlockSpec((NC, blk, d), lambda i: (0, i, 0)),
            pl.BlockSpec((blk, d), lambda i: (i, 0)),
            pl.BlockSpec((1, d), lambda i: (0, 0)),
        ],
        out_specs=pl.BlockSpec((blk, d), lambda i: (i, 0)),
        out_shape=jax.ShapeDtypeStruct((n, d), F32),
    )(degp, p, hs, b)


def kernel(x, edge_index, W1, b1, W2, b2, W3, b3):
    n, d = x.shape
    e = edge_index.shape[1]
    nch = (e + NW * C - 1) // (NW * C)
    ep = NW * nch * C
    n_deg = ((n + NS * 128 - 1) // (NS * 128)) * (NS * 128)
    n_acc = n_deg

    trash = n + jnp.arange(ep - e, dtype=jnp.int32) % (n_acc - n)
    src = jnp.pad(edge_index[0].astype(jnp.int32), (0, ep - e)
                  ).reshape(NW, nch, C)
    dst = jnp.concatenate([edge_index[1].astype(jnp.int32), trash]
                          ).reshape(NW, nch, C)
    ones_r = jnp.ones((C,), F32)
    zdeg = jnp.zeros((n_deg // NS,), F32)
    zmsg = jnp.zeros((n_acc // NS, d), F32)

    deg_fn = _make_deg(n_deg, nch)
    msg_fn = _make_msg(n_acc, d, nch)

    xp = jnp.pad(x, ((0, n_deg - n), (0, 0)))
    degp = deg_fn(dst, ones_r, zdeg).reshape(NC, n_deg, 1)
    hs1 = _tc_first(degp, xp, W1)
    p1 = msg_fn(hs1, src, dst, zmsg)
    hs2 = _tc_mid(degp, p1, hs1, b1.reshape(1, -1), W2)
    p2 = msg_fn(hs2, src, dst, zmsg)
    hs3 = _tc_mid(degp, p2, hs2, b2.reshape(1, -1), W3)
    p3 = msg_fn(hs3, src, dst, zmsg)
    return _tc_final(degp, p3, hs3, b3.reshape(1, -1))[:n]


# speedup vs baseline: 1.7011x; 1.0363x over previous
import functools

import jax
import jax.numpy as jnp
from jax import lax
from jax.experimental import pallas as pl
from jax.experimental.pallas import tpu as pltpu
from jax.experimental.pallas import tpu_sc as plsc

F32 = jnp.float32
NC = 2
NS = 16
NW = NC * NS
C = 125


def _sc_mesh():
    return plsc.VectorSubcoreMesh(
        core_axis_name="c", subcore_axis_name="s",
        num_cores=NC, num_subcores=NS)


def _make_deg(n, nch):
    rpt = n // NS

    @functools.partial(
        pl.kernel,
        out_type=jax.ShapeDtypeStruct((NC, n), F32),
        mesh=_sc_mesh(),
        scratch_types=[
            pltpu.VMEM((nch, C), jnp.int32),
            pltpu.VMEM((C,), F32),
            pltpu.VMEM_SHARED((n,), F32),
        ],
    )
    def deg_kernel(dst_hbm, ones_hbm, zero_hbm, out_hbm, dstv, onesv, acc):
        cid = lax.axis_index("c")
        sid = lax.axis_index("s")
        wid = sid * NC + cid
        pltpu.sync_copy(dst_hbm.at[wid], dstv)
        pltpu.sync_copy(ones_hbm, onesv)
        pltpu.sync_copy(zero_hbm, acc.at[pl.ds(sid * rpt, rpt)])
        plsc.subcore_barrier()

        def step(j, carry):
            pltpu.sync_copy(onesv, acc.at[dstv.at[j]], add=True)
            return carry
        lax.fori_loop(0, nch, step, 0)

        plsc.subcore_barrier()
        pltpu.sync_copy(acc.at[pl.ds(sid * rpt, rpt)],
                        out_hbm.at[cid].at[pl.ds(sid * rpt, rpt)])

    return deg_kernel


def _make_msg(n, d, nch):
    rpt = n // NS

    @functools.partial(
        pl.kernel,
        out_type=jax.ShapeDtypeStruct((NC, n, d), F32),
        mesh=_sc_mesh(),
        scratch_types=[
            pltpu.VMEM((nch, C), jnp.int32),
            pltpu.VMEM((nch, C), jnp.int32),
            pltpu.VMEM((C, d), F32),
            pltpu.VMEM_SHARED((n, d), F32),
            pltpu.SemaphoreType.DMA,
        ],
    )
    def msg_kernel(hs_hbm, src_hbm, dst_hbm, zero_hbm, out_hbm,
                   srcv, dstv, rowsv, acc, sem):
        cid = lax.axis_index("c")
        sid = lax.axis_index("s")
        wid = sid * NC + cid
        pltpu.sync_copy(src_hbm.at[wid], srcv)
        pltpu.async_copy(hs_hbm.at[srcv.at[0]], rowsv, sem)
        pltpu.sync_copy(dst_hbm.at[wid], dstv)
        pltpu.sync_copy(zero_hbm, acc.at[pl.ds(sid * rpt, rpt)])
        plsc.subcore_barrier()

        def step(j, carry):
            pltpu.make_async_copy(hs_hbm.at[srcv.at[j]], rowsv, sem).wait()
            pltpu.sync_copy(rowsv, acc.at[dstv.at[j]], add=True)

            @pl.when(j + 1 < nch)
            def _():
                pltpu.async_copy(hs_hbm.at[srcv.at[j + 1]], rowsv, sem)
            return carry
        lax.fori_loop(0, nch, step, 0)

        plsc.subcore_barrier()
        pltpu.sync_copy(acc.at[pl.ds(sid * rpt, rpt)],
                        out_hbm.at[cid].at[pl.ds(sid * rpt, rpt)])

    return msg_kernel


def _dinv_of(deg_blk):
    return lax.rsqrt(deg_blk[0] + deg_blk[1] + 1.0)


def _tc_first(degp, x, w):
    n, d = x.shape
    blk = n // 10

    def body(deg_ref, x_ref, w_ref, o_ref):
        dinv = _dinv_of(deg_ref[...])
        o_ref[...] = jnp.dot(x_ref[...], w_ref[...],
                             preferred_element_type=F32) * dinv

    return pl.pallas_call(
        body,
        grid=(n // blk,),
        in_specs=[
            pl.BlockSpec((NC, blk, 1), lambda i: (0, i, 0)),
            pl.BlockSpec((blk, d), lambda i: (i, 0)),
            pl.BlockSpec((d, d), lambda i: (0, 0)),
        ],
        out_specs=pl.BlockSpec((blk, d), lambda i: (i, 0)),
        out_shape=jax.ShapeDtypeStruct((n, d), F32),
    )(degp, x, w)


def _tc_mid(degp, p, hs, b, w):
    n, d = hs.shape
    blk = n // 10

    def body(deg_ref, p_ref, hs_ref, b_ref, w_ref, o_ref):
        dinv = _dinv_of(deg_ref[...])
        pp = p_ref[...]
        o = jnp.maximum((pp[0] + pp[1] + hs_ref[...]) * dinv + b_ref[...], 0.0)
        o_ref[...] = jnp.dot(o, w_ref[...], preferred_element_type=F32) * dinv

    return pl.pallas_call(
        body,
        grid=(n // blk,),
        in_specs=[
            pl.BlockSpec((NC, blk, 1), lambda i: (0, i, 0)),
            pl.BlockSpec((NC, blk, d), lambda i: (0, i, 0)),
            pl.BlockSpec((blk, d), lambda i: (i, 0)),
            pl.BlockSpec((1, d), lambda i: (0, 0)),
            pl.BlockSpec((d, d), lambda i: (0, 0)),
        ],
        out_specs=pl.BlockSpec((blk, d), lambda i: (i, 0)),
        out_shape=jax.ShapeDtypeStruct((n, d), F32),
    )(degp, p, hs, b, w)


def _tc_final(degp, p, hs, b):
    n, d = hs.shape
    blk = n // 10

    def body(deg_ref, p_ref, hs_ref, b_ref, o_ref):
        dinv = _dinv_of(deg_ref[...])
        pp = p_ref[...]
        o_ref[...] = jnp.maximum(
            (pp[0] + pp[1] + hs_ref[...]) * dinv + b_ref[...], 0.0)

    return pl.pallas_call(
        body,
        grid=(n // blk,),
        in_specs=[
            pl.BlockSpec((NC, blk, 1), lambda i: (0, i, 0)),
            pl.BlockSpec((NC, blk, d), lambda i: (0, i, 0)),
            pl.BlockSpec((blk, d), lambda i: (i, 0)),
            pl.BlockSpec((1, d), lambda i: (0, 0)),
        ],
        out_specs=pl.BlockSpec((blk, d), lambda i: (i, 0)),
        out_shape=jax.ShapeDtypeStruct((n, d), F32),
    )(degp, p, hs, b)


def kernel(x, edge_index, W1, b1, W2, b2, W3, b3):
    n, d = x.shape
    e = edge_index.shape[1]
    nch = (e + NW * C - 1) // (NW * C)
    ep = NW * nch * C
    n_deg = ((n + NS * 128 - 1) // (NS * 128)) * (NS * 128)
    n_acc = n_deg

    trash = n + jnp.arange(ep - e, dtype=jnp.int32) % (n_acc - n)
    src = jnp.pad(edge_index[0].astype(jnp.int32), (0, ep - e)
                  ).reshape(NW, nch, C)
    dst = jnp.concatenate([edge_index[1].astype(jnp.int32), trash]
                          ).reshape(NW, nch, C)
    ones_r = jnp.ones((C,), F32)
    zdeg = jnp.zeros((n_deg // NS,), F32)
    zmsg = jnp.zeros((n_acc // NS, d), F32)

    deg_fn = _make_deg(n_deg, nch)
    msg_fn = _make_msg(n_acc, d, nch)

    degp = deg_fn(dst, ones_r, zdeg).reshape(NC, n_deg, 1)
    hs1 = _tc_first(degp, x, W1)
    p1 = msg_fn(hs1, src, dst, zmsg)
    hs2 = _tc_mid(degp, p1, hs1, b1.reshape(1, -1), W2)
    p2 = msg_fn(hs2, src, dst, zmsg)
    hs3 = _tc_mid(degp, p2, hs2, b2.reshape(1, -1), W3)
    p3 = msg_fn(hs3, src, dst, zmsg)
    return _tc_final(degp, p3, hs3, b3.reshape(1, -1))
